# baseline (speedup 1.0000x reference)
import functools

import jax
import jax.numpy as jnp
from jax.experimental import pallas as pl

_EPS = 1e-10
_CUTPER = 0.2


def _rows_body(idx_ref, rm_ref, bnd_ref, sum_ref, val_ref, sb_ref):
    blk = rm_ref[...]
    idx = idx_ref[0, 0, :]
    cols = jax.lax.broadcasted_iota(jnp.int32, blk.shape, 1)
    m = cols == idx[:, None]
    sum_ref[0, 0, :] = jnp.sum(blk, axis=1)
    val_ref[0, 0, :] = jnp.sum(jnp.where(m, blk, 0.0), axis=1)
    bnd = bnd_ref[0]
    sb_ref[0, 0, :] = jnp.sum(jnp.where(m, bnd, 0.0), axis=1)


def _loss_body(sr_ref, vr_ref, br_ref, sc_ref, vc_ref, bc_ref, out_ref, *, k):
    x_row = br_ref[...] * -jnp.log(_EPS + vr_ref[...] / sr_ref[...])
    x_col = bc_ref[...] * -jnp.log(_EPS + vc_ref[...] / sc_ref[...])
    le = (x_row <= x_col).astype(jnp.float32)
    cnt = jnp.sum(le, axis=1, keepdims=True)
    elig = cnt >= jnp.float32(k + 1)
    thr = jnp.min(jnp.where(elig, x_col, jnp.inf), keepdims=True)
    m = x_row > thr
    numer = jnp.sum(jnp.where(m, x_row, 0.0), axis=1, keepdims=True)
    denom = 1.0 + jnp.sum(jnp.where(m, br_ref[...], 0.0), axis=1, keepdims=True)
    out_ref[...] = numer / denom


def kernel(response_map, source_feature_1d_locations, boundaries):
    B, S, H, W = response_map.shape
    HW = H * W
    N = B * S
    R = 64
    NB = N // R

    rm2 = response_map.reshape(N, HW)
    bnd3 = boundaries.reshape(B, 1, HW)
    idx3 = source_feature_1d_locations.astype(jnp.int32).reshape(NB, 1, R)

    sums, vals, sb = pl.pallas_call(
        _rows_body,
        grid=(NB,),
        in_specs=[
            pl.BlockSpec((1, 1, R), lambda i: (i, 0, 0)),
            pl.BlockSpec((R, HW), lambda i: (i, 0)),
            pl.BlockSpec((1, 1, HW), lambda i: (i // (S // R), 0, 0)),
        ],
        out_specs=[
            pl.BlockSpec((1, 1, R), lambda i: (i, 0, 0)),
            pl.BlockSpec((1, 1, R), lambda i: (i, 0, 0)),
            pl.BlockSpec((1, 1, R), lambda i: (i, 0, 0)),
        ],
        out_shape=[
            jax.ShapeDtypeStruct((NB, 1, R), jnp.float32),
            jax.ShapeDtypeStruct((NB, 1, R), jnp.float32),
            jax.ShapeDtypeStruct((NB, 1, R), jnp.float32),
        ],
    )(idx3, rm2, bnd3)

    k = int(N * _CUTPER)
    sr = sums.reshape(1, N)
    vr = vals.reshape(1, N)
    br = sb.reshape(1, N)

    loss = pl.pallas_call(
        functools.partial(_loss_body, k=k),
        out_shape=jax.ShapeDtypeStruct((1, 1), jnp.float32),
    )(sr, vr, br, sr.reshape(N, 1), vr.reshape(N, 1), br.reshape(N, 1))

    return loss.reshape(())


# speedup vs baseline: 1.3210x; 1.3210x over previous
import functools

import jax
import jax.numpy as jnp
from jax.experimental import pallas as pl

_EPS = 1e-10
_CUTPER = 0.2


def _rows_body(idx_ref, rm_ref, bnd_ref, sum_ref, val_ref, sb_ref, *, W):
    idxv = idx_ref[0, 0, 0, :]
    blk = rm_ref[0]
    bnd = bnd_ref[0]
    R, H, _ = blk.shape
    w_t = idxv % W
    h_t = idxv // W
    lane = jax.lax.broadcasted_iota(jnp.int32, (R, 1, W), 2)
    mw = lane == w_t[:, None, None]
    wsum = jnp.sum(blk, axis=2)
    tv = jnp.sum(jnp.where(mw, blk, 0.0), axis=2)
    tb = jnp.sum(jnp.where(mw, jnp.broadcast_to(bnd, blk.shape), 0.0), axis=2)
    hh = jax.lax.broadcasted_iota(jnp.int32, (R, H), 1)
    mh = hh == h_t[:, None]
    sum_ref[0, 0, 0, :] = jnp.sum(wsum, axis=1)
    val_ref[0, 0, 0, :] = jnp.sum(jnp.where(mh, tv, 0.0), axis=1)
    sb_ref[0, 0, 0, :] = jnp.sum(jnp.where(mh, tb, 0.0), axis=1)


def _loss_body(sr_ref, vr_ref, br_ref, sc_ref, vc_ref, bc_ref, out_ref, *, k):
    x_row = br_ref[...] * -jnp.log(_EPS + vr_ref[...] / sr_ref[...])
    x_col = bc_ref[...] * -jnp.log(_EPS + vc_ref[...] / sc_ref[...])
    le = (x_row <= x_col).astype(jnp.float32)
    cnt = jnp.sum(le, axis=1, keepdims=True)
    elig = cnt >= jnp.float32(k + 1)
    thr = jnp.min(jnp.where(elig, x_col, jnp.inf), keepdims=True)
    m = x_row > thr
    numer = jnp.sum(jnp.where(m, x_row, 0.0), axis=1, keepdims=True)
    denom = 1.0 + jnp.sum(jnp.where(m, br_ref[...], 0.0), axis=1, keepdims=True)
    out_ref[...] = numer / denom


def kernel(response_map, source_feature_1d_locations, boundaries):
    B, S, H, W = response_map.shape
    N = B * S
    R = 16
    NSB = S // R

    idx4 = source_feature_1d_locations.astype(jnp.int32).reshape(B, NSB, 1, R)

    sums, vals, sb = pl.pallas_call(
        functools.partial(_rows_body, W=W),
        grid=(B, NSB),
        in_specs=[
            pl.BlockSpec((1, 1, 1, R), lambda b, j: (b, j, 0, 0)),
            pl.BlockSpec((1, R, H, W), lambda b, j: (b, j, 0, 0)),
            pl.BlockSpec((1, 1, H, W), lambda b, j: (b, 0, 0, 0)),
        ],
        out_specs=[
            pl.BlockSpec((1, 1, 1, R), lambda b, j: (b, j, 0, 0)),
            pl.BlockSpec((1, 1, 1, R), lambda b, j: (b, j, 0, 0)),
            pl.BlockSpec((1, 1, 1, R), lambda b, j: (b, j, 0, 0)),
        ],
        out_shape=[
            jax.ShapeDtypeStruct((B, NSB, 1, R), jnp.float32),
            jax.ShapeDtypeStruct((B, NSB, 1, R), jnp.float32),
            jax.ShapeDtypeStruct((B, NSB, 1, R), jnp.float32),
        ],
    )(idx4, response_map, boundaries)

    k = int(N * _CUTPER)
    sr = sums.reshape(1, N)
    vr = vals.reshape(1, N)
    br = sb.reshape(1, N)

    loss = pl.pallas_call(
        functools.partial(_loss_body, k=k),
        out_shape=jax.ShapeDtypeStruct((1, 1), jnp.float32),
    )(sr, vr, br, sr.reshape(N, 1), vr.reshape(N, 1), br.reshape(N, 1))

    return loss.reshape(())


# speedup vs baseline: 1.5391x; 1.1651x over previous
import functools

import jax
import jax.numpy as jnp
from jax import lax
from jax.experimental import pallas as pl
from jax.experimental.pallas import tpu as pltpu
from jax.experimental.pallas import tpu_sc as plsc

_EPS = 1e-10
_CUTPER = 0.2

_NC = 2
_NS = 16
_NW = _NC * _NS


def _rows_body(idx_ref, rm_ref, sum_ref, val_ref, *, W):
    idxv = idx_ref[0, 0, 0, :]
    blk = rm_ref[0]
    R, H, _ = blk.shape
    w_t = idxv % W
    h_t = idxv // W
    lane = jax.lax.broadcasted_iota(jnp.int32, (R, 1, W), 2)
    mw = lane == w_t[:, None, None]
    wsum = jnp.sum(blk, axis=2)
    tv = jnp.sum(jnp.where(mw, blk, 0.0), axis=2)
    hh = jax.lax.broadcasted_iota(jnp.int32, (R, H), 1)
    mh = hh == h_t[:, None]
    sum_ref[0, 0, 0, :] = jnp.sum(wsum, axis=1)
    val_ref[0, 0, 0, :] = jnp.sum(jnp.where(mh, tv, 0.0), axis=1)


def _sc_gather_body(bnd_ref, idx_ref, out_ref, idx_v, gidx_v, out_v, sem,
                    *, S, HW, PW):
    wid = lax.axis_index("s") * _NC + lax.axis_index("c")
    base = wid * PW
    pltpu.sync_copy(idx_ref.at[pl.ds(base, PW)], idx_v)
    off = (base // S) * HW
    for j in range(PW // 16):
        gidx_v[pl.ds(j * 16, 16)] = idx_v[pl.ds(j * 16, 16)] + off
    pltpu.async_copy(bnd_ref.at[gidx_v], out_v, sem).wait()
    pltpu.sync_copy(out_v, out_ref.at[pl.ds(base, PW)])


def _loss_body(sr_ref, vr_ref, br_ref, sc_ref, vc_ref, bc_ref, out_ref, *, k):
    x_row = br_ref[...] * -jnp.log(_EPS + vr_ref[...] / sr_ref[...])
    x_col = bc_ref[...] * -jnp.log(_EPS + vc_ref[...] / sc_ref[...])
    le = (x_row <= x_col).astype(jnp.float32)
    cnt = jnp.sum(le, axis=1, keepdims=True)
    elig = cnt >= jnp.float32(k + 1)
    thr = jnp.min(jnp.where(elig, x_col, jnp.inf), keepdims=True)
    m = x_row > thr
    numer = jnp.sum(jnp.where(m, x_row, 0.0), axis=1, keepdims=True)
    denom = 1.0 + jnp.sum(jnp.where(m, br_ref[...], 0.0), axis=1, keepdims=True)
    out_ref[...] = numer / denom


def kernel(response_map, source_feature_1d_locations, boundaries):
    B, S, H, W = response_map.shape
    HW = H * W
    N = B * S
    R = 64
    NSB = S // R
    PW = N // _NW

    idx_i32 = source_feature_1d_locations.astype(jnp.int32)
    idx4 = idx_i32.reshape(B, NSB, 1, R)

    sums, vals = pl.pallas_call(
        functools.partial(_rows_body, W=W),
        grid=(B, NSB),
        in_specs=[
            pl.BlockSpec((1, 1, 1, R), lambda b, j: (b, j, 0, 0)),
            pl.BlockSpec((1, R, H, W), lambda b, j: (b, j, 0, 0)),
        ],
        out_specs=[
            pl.BlockSpec((1, 1, 1, R), lambda b, j: (b, j, 0, 0)),
            pl.BlockSpec((1, 1, 1, R), lambda b, j: (b, j, 0, 0)),
        ],
        out_shape=[
            jax.ShapeDtypeStruct((B, NSB, 1, R), jnp.float32),
            jax.ShapeDtypeStruct((B, NSB, 1, R), jnp.float32),
        ],
    )(idx4, response_map)

    sb_flat = pl.kernel(
        functools.partial(_sc_gather_body, S=S, HW=HW, PW=PW),
        out_type=jax.ShapeDtypeStruct((N,), jnp.float32),
        mesh=plsc.VectorSubcoreMesh(core_axis_name="c", subcore_axis_name="s",
                                    num_cores=_NC, num_subcores=_NS),
        scratch_types=[
            pltpu.VMEM((PW,), jnp.int32),
            pltpu.VMEM((PW,), jnp.int32),
            pltpu.VMEM((PW,), jnp.float32),
            pltpu.SemaphoreType.DMA,
        ],
    )(boundaries.reshape(B * HW), idx_i32.reshape(N))

    k = int(N * _CUTPER)
    sr = sums.reshape(1, N)
    vr = vals.reshape(1, N)
    br = sb_flat.reshape(1, N)

    loss = pl.pallas_call(
        functools.partial(_loss_body, k=k),
        out_shape=jax.ShapeDtypeStruct((1, 1), jnp.float32),
    )(sr, vr, br, sr.reshape(N, 1), vr.reshape(N, 1), br.reshape(N, 1))

    return loss.reshape(())
